# baseline (speedup 1.0000x reference)
import functools

import jax
import jax.numpy as jnp
from jax import lax
from jax.experimental import pallas as pl
from jax.experimental.pallas import tpu as pltpu
from jax.experimental.pallas import tpu_sc as plsc

_BATCH, _HIST, _DIM = 4096, 200, 32
_B = _BATCH * _HIST

_info = plsc.get_sparse_core_info()
_NC, _NS = _info.num_cores, _info.num_subcores
_NW = _NC * _NS
_BPW = _B // _NW
_CH = 800
_NBUF = 4
_NCHUNK = _BPW // _CH
_NGROUP = _NCHUNK // _NBUF

_mesh = plsc.VectorSubcoreMesh(core_axis_name="c", subcore_axis_name="s")


@functools.partial(
    pl.kernel,
    mesh=_mesh,
    out_type=jax.ShapeDtypeStruct((_B, _DIM), jnp.float32),
    scratch_types=[
        *[pltpu.VMEM((_CH,), jnp.int32) for _ in range(_NBUF)],
        *[pltpu.VMEM((_CH, _DIM), jnp.float32) for _ in range(_NBUF)],
        pltpu.SemaphoreType.DMA,
        pltpu.SemaphoreType.DMA,
    ],
    compiler_params=pltpu.CompilerParams(use_tc_tiling_on_sc=False),
)
def _gather(idx_hbm, table_hbm, out_hbm, i0, i1, i2, i3, r0, r1, r2, r3,
            gat_sem, out_sem):
    wid = lax.axis_index("s") * _NC + lax.axis_index("c")
    base = wid * _BPW
    idx_v = [i0, i1, i2, i3]
    rows_v = [r0, r1, r2, r3]

    def load_idx(c, b):
        pltpu.sync_copy(idx_hbm.at[pl.ds(base + c * _CH, _CH)], idx_v[b])

    def fire_gather(b):
        pltpu.async_copy(table_hbm.at[idx_v[b]], rows_v[b], gat_sem)

    def wait_gather(b):
        pltpu.make_async_copy(table_hbm.at[idx_v[b]], rows_v[b], gat_sem).wait()

    def fire_out(c, b):
        pltpu.async_copy(rows_v[b], out_hbm.at[pl.ds(base + c * _CH, _CH)], out_sem)

    def wait_out(c, b):
        pltpu.make_async_copy(
            rows_v[b], out_hbm.at[pl.ds(base + c * _CH, _CH)], out_sem
        ).wait()

    for b in range(_NBUF):
        load_idx(b, b)
        fire_gather(b)

    def group(o, carry):
        for b in range(_NBUF):
            c = o * _NBUF + b
            wait_gather(b)
            fire_out(c, b)
            wait_out(c, b)
            load_idx(c + _NBUF, b)
            fire_gather(b)
        return carry

    lax.fori_loop(0, _NGROUP - 1, group, 0)

    for b in range(_NBUF):
        c = (_NGROUP - 1) * _NBUF + b
        wait_gather(b)
        fire_out(c, b)
        wait_out(c, b)


def kernel(inputs, table):
    flat = inputs.reshape(-1).astype(jnp.int32)
    out = _gather(flat, table)
    return out.reshape(_BATCH, _HIST, _DIM)
